# baseline (speedup 1.0000x reference)
import functools

import jax
import jax.numpy as jnp
from jax import lax
from jax.experimental import pallas as pl
from jax.experimental.pallas import tpu as pltpu
from jax.experimental.pallas import tpu_sc as plsc

B = 1024
D = 64
X_MAX = 100.0
ALPHA = 0.75

NC = 2
NS = 16
NW = NC * NS
BPW = B // NW
PK = 128


def _sc_gather(idx_i_hbm, idx_j_hbm, emb_i_hbm, emb_j_hbm, bi_hbm, bj_hbm,
               packed_out, bsum_out,
               ii_v, ij_v, ri_v, rj_v, bi_v, bj_v, bs_v, p_v, sem):
    wid = lax.axis_index("s") * NC + lax.axis_index("c")
    base = wid * BPW
    pltpu.sync_copy(idx_i_hbm.at[pl.ds(base, BPW)], ii_v)
    pltpu.sync_copy(idx_j_hbm.at[pl.ds(base, BPW)], ij_v)
    cp1 = pltpu.async_copy(emb_i_hbm.at[ii_v], ri_v, sem)
    cp2 = pltpu.async_copy(emb_j_hbm.at[ij_v], rj_v, sem)
    cp3 = pltpu.async_copy(bi_hbm.at[ii_v], bi_v, sem)
    cp4 = pltpu.async_copy(bj_hbm.at[ij_v], bj_v, sem)
    cp1.wait()
    cp2.wait()
    cp3.wait()
    cp4.wait()

    def row(r, _):
        for c in range(D // 16):
            p_v[r, pl.ds(c * 16, 16)] = (
                ri_v[r, pl.ds(c * 16, 16)] * rj_v[r, pl.ds(c * 16, 16)])
        return 0

    lax.fori_loop(0, BPW, row, 0)
    for g in range(BPW // 16):
        sl = pl.ds(g * 16, 16)
        bs_v[sl] = bi_v[sl] + bj_v[sl]

    pltpu.sync_copy(p_v, packed_out.at[pl.ds(base, BPW), pl.ds(0, D)])
    pltpu.sync_copy(bs_v, bsum_out.at[pl.ds(base, BPW)])


_sc_kernel = functools.partial(
    pl.kernel,
    out_type=(
        jax.ShapeDtypeStruct((B, PK), jnp.float32),
        jax.ShapeDtypeStruct((B,), jnp.float32),
    ),
    mesh=plsc.VectorSubcoreMesh(core_axis_name="c", subcore_axis_name="s"),
    compiler_params=pltpu.CompilerParams(
        needs_layout_passes=False, use_tc_tiling_on_sc=False),
    scratch_types=[
        pltpu.VMEM((BPW,), jnp.int32),
        pltpu.VMEM((BPW,), jnp.int32),
        pltpu.VMEM((BPW, D), jnp.float32),
        pltpu.VMEM((BPW, D), jnp.float32),
        pltpu.VMEM((BPW,), jnp.float32),
        pltpu.VMEM((BPW,), jnp.float32),
        pltpu.VMEM((BPW,), jnp.float32),
        pltpu.VMEM((BPW, D), jnp.float32),
        pltpu.SemaphoreType.DMA,
    ],
)(_sc_gather)


ROW_BLK = 128


def _tc_outer(xij_ref, packed_ref, bsum_ref, out_ref):
    xf = xij_ref[...].astype(jnp.float32)
    t = jnp.log(xf)
    fx = jnp.where(xf >= X_MAX, jnp.float32(1.0),
                   jnp.exp(ALPHA * jnp.log(xf * (1.0 / X_MAX))))
    prod = packed_ref[:, :D]
    s = jnp.sum(prod, axis=1, keepdims=True) + bsum_ref[...]
    diff = s - t
    out_ref[...] = fx * diff * diff


def kernel(x, emb_i, emb_j, bi, bj):
    idx_i = x[:, 0]
    idx_j = x[:, 1]
    xij2 = x[:, 2].reshape(1, B)

    packed, bsum = _sc_kernel(idx_i, idx_j, emb_i, emb_j,
                              bi.reshape(-1), bj.reshape(-1))

    out2 = pl.pallas_call(
        _tc_outer,
        grid=(B // ROW_BLK,),
        in_specs=[
            pl.BlockSpec((1, B), lambda i: (0, 0)),
            pl.BlockSpec((ROW_BLK, PK), lambda i: (i, 0)),
            pl.BlockSpec((ROW_BLK, 1), lambda i: (i, 0)),
        ],
        out_specs=pl.BlockSpec((ROW_BLK, B), lambda i: (i, 0)),
        out_shape=jax.ShapeDtypeStruct((B, B), jnp.float32),
    )(xij2, packed, bsum.reshape(B, 1))

    return out2.reshape(B, 1, B)
